# Initial kernel scaffold; baseline (speedup 1.0000x reference)
import jax
import jax.numpy as jnp
from jax.experimental import pallas as pl


def kernel(node_feat, node_sh, edge_index, diff_pos, diff_vel, radial, W_msg1, b_msg1, W_msg2, b_msg2, W_pos1, b_pos1, W_pos2, b_pos2, W_vel1, b_vel1, W_vel2, b_vel2, W_node1, b_node1, W_node2, b_node2, W_sh1, b_sh1, W_sh2, b_sh2):
    raise NotImplementedError("write your pallas kernel here")


# speedup vs baseline: 2.9992x; 2.9992x over previous
import functools

import jax
import jax.numpy as jnp
from jax import lax
from jax.experimental import pallas as pl
from jax.experimental.pallas import tpu as pltpu
from jax.experimental.pallas import tpu_sc as plsc

N = 10000
E = 320000
H = 128

NC = 2
NS = 16
NW = NC * NS
EPW = E // NW
CE = 80
NCHUNK = EPW // CE
NACC = 10240
RPT = NACC // NS
EPT = E // NS
NCHUNK_S = EPT // CE

BE = 2000
BN = 1000

f32 = jnp.float32


def _silu(x):
    return x * (1.0 / (1.0 + jnp.exp(-x)))


def _sc_gather(feat, shp, row, col):
    mesh = plsc.VectorSubcoreMesh(core_axis_name="c", subcore_axis_name="s")

    @functools.partial(
        pl.kernel,
        out_type=(
            jax.ShapeDtypeStruct((E, H), f32),
            jax.ShapeDtypeStruct((E, H), f32),
            jax.ShapeDtypeStruct((E, H), f32),
            jax.ShapeDtypeStruct((E, H), f32),
        ),
        mesh=mesh,
        scratch_types=(
            pltpu.VMEM((CE,), jnp.int32),
            pltpu.VMEM((CE,), jnp.int32),
            pltpu.VMEM((CE, H), f32),
            pltpu.VMEM((CE, H), f32),
            pltpu.SemaphoreType.DMA,
        ),
    )
    def gk(feat_hbm, shp_hbm, row_hbm, col_hbm,
           fr_hbm, fc_hbm, sr_hbm, sc_hbm,
           idxr_v, idxc_v, fbuf, sbuf, sem):
        cid = lax.axis_index("c")
        sid = lax.axis_index("s")
        wid = sid * NC + cid

        def body(t, carry):
            base = wid * EPW + t * CE
            pltpu.sync_copy(row_hbm.at[pl.ds(base, CE)], idxr_v)
            pltpu.sync_copy(col_hbm.at[pl.ds(base, CE)], idxc_v)
            pltpu.async_copy(feat_hbm.at[idxr_v], fbuf, sem).wait()
            pltpu.sync_copy(fbuf, fr_hbm.at[pl.ds(base, CE)])
            pltpu.async_copy(feat_hbm.at[idxc_v], fbuf, sem).wait()
            pltpu.sync_copy(fbuf, fc_hbm.at[pl.ds(base, CE)])
            pltpu.async_copy(shp_hbm.at[idxr_v], sbuf, sem).wait()
            pltpu.sync_copy(sbuf, sr_hbm.at[pl.ds(base, CE)])
            pltpu.async_copy(shp_hbm.at[idxc_v], sbuf, sem).wait()
            pltpu.sync_copy(sbuf, sc_hbm.at[pl.ds(base, CE)])
            return carry

        lax.fori_loop(0, NCHUNK, body, 0)

    return gk(feat, shp, row, col)


def _sc_scatter(payload, row, zf):
    mesh = plsc.VectorSubcoreMesh(core_axis_name="c", subcore_axis_name="s")

    @functools.partial(
        pl.kernel,
        out_type=jax.ShapeDtypeStruct((NC, NACC, H), f32),
        mesh=mesh,
        scratch_types=(
            pltpu.VMEM((CE,), jnp.int32),
            pltpu.VMEM((CE, H), f32),
            pltpu.VMEM_SHARED((NACC, H), f32),
            pltpu.SemaphoreType.DMA,
        ),
    )
    def sk(p_hbm, row_hbm, zf_hbm, out_hbm, idx_v, pbuf, acc, sem):
        cid = lax.axis_index("c")
        sid = lax.axis_index("s")
        r0 = sid * RPT
        pltpu.sync_copy(zf_hbm.at[pl.ds(r0, RPT)], acc.at[pl.ds(r0, RPT)])
        plsc.subcore_barrier()

        def body(t, carry):
            base = sid * EPT + t * CE
            pltpu.sync_copy(row_hbm.at[pl.ds(base, CE)], idx_v)
            pltpu.sync_copy(p_hbm.at[cid, pl.ds(base, CE)], pbuf)
            pltpu.sync_copy(pbuf, acc.at[idx_v], add=True)
            return carry

        lax.fori_loop(0, NCHUNK_S, body, 0)
        plsc.subcore_barrier()
        pltpu.sync_copy(acc.at[pl.ds(r0, RPT)], out_hbm.at[cid, pl.ds(r0, RPT)])

    return sk(payload, row, zf)


def _edge_body(fr, fc, sr, sc_, rad, dpv,
               w1r, w1c, w1rad, w1ip, b1, w2, b2, wh1, bh1, wh2, bh2,
               p_o):
    srv = sr[...]
    scv = sc_[...]
    p = srv * scv
    ip0 = p[:, 0:1]
    ip1 = jnp.sum(p[:, 1:4], axis=1, keepdims=True)
    ip2 = jnp.sum(p[:, 4:9], axis=1, keepdims=True)
    pre = (jnp.dot(fr[...], w1r[...], preferred_element_type=f32)
           + jnp.dot(fc[...], w1c[...], preferred_element_type=f32)
           + jnp.dot(rad[...], w1rad[...], preferred_element_type=f32)
           + b1[...])
    pre = pre + ip0 * w1ip[0:1, :] + ip1 * w1ip[1:2, :] + ip2 * w1ip[2:3, :]
    h = _silu(pre)
    msg = _silu(jnp.dot(h, w2[...], preferred_element_type=f32) + b2[...])
    gh = _silu(jnp.dot(msg, wh1[...], preferred_element_type=f32) + bh1[...])
    g = jnp.dot(gh, wh2[...], preferred_element_type=f32) + bh2[...]
    dp = dpv[:, 0:3]
    dv = dpv[:, 3:6]
    evp = g[:, 0:1] * dp + g[:, 1:2] * dv
    evv = g[:, 2:3] * dv + g[:, 3:4] * dp
    dsh = srv - scv
    gsh = jnp.concatenate(
        [g[:, 4:5] * dsh[:, 0:1], g[:, 5:6] * dsh[:, 1:4], g[:, 6:7] * dsh[:, 4:9]],
        axis=1)
    ones = jnp.full((evp.shape[0], 1), 1.0, f32)
    zpad = jnp.zeros((evp.shape[0], H - 16), f32)
    p_o[0] = msg
    p_o[1] = jnp.concatenate([evp, evv, gsh, ones, zpad], axis=1)


def _tc_edge(fr, fc, sr, sc_, rad, dpv, *ws):
    def im_e(i):
        return (i, 0)

    def im_w(i):
        return (0, 0)

    in_specs = [
        pl.BlockSpec((BE, H), im_e),
        pl.BlockSpec((BE, H), im_e),
        pl.BlockSpec((BE, H), im_e),
        pl.BlockSpec((BE, H), im_e),
        pl.BlockSpec((BE, 16), im_e),
        pl.BlockSpec((BE, 8), im_e),
    ] + [pl.BlockSpec(w.shape, im_w) for w in ws]
    return pl.pallas_call(
        _edge_body,
        grid=(E // BE,),
        in_specs=in_specs,
        out_specs=pl.BlockSpec((2, BE, H), lambda i: (0, i, 0)),
        out_shape=jax.ShapeDtypeStruct((2, E, H), f32),
        compiler_params=pltpu.CompilerParams(
            dimension_semantics=("arbitrary",)),
    )(fr, fc, sr, sc_, rad, dpv, *ws)


def _node_body(nf, nsh, pr, wn1a, wn1b, bn1, wn2, bn2,
               nf_o, nsh_o, pos_o, vel_o):
    ms = pr[0]
    sm = pr[1]
    inv = 1.0 / jnp.maximum(sm[:, 15:16], 1.0)
    msg_agg = ms * inv
    pos_o[...] = sm[:, 0:3] * inv
    vel_o[...] = sm[:, 3:6] * inv
    nsh_o[...] = nsh[...] + sm[:, 6:15] * inv
    hh = _silu(jnp.dot(nf[...], wn1a[...], preferred_element_type=f32)
               + jnp.dot(msg_agg, wn1b[...], preferred_element_type=f32)
               + bn1[...])
    nf_o[...] = jnp.dot(hh, wn2[...], preferred_element_type=f32) + bn2[...]


def _tc_node(nf, nsh, pagg, *ws):
    def im_n(i):
        return (i, 0)

    def im_p(i):
        return (0, i, 0)

    def im_w(i):
        return (0, 0)

    in_specs = [
        pl.BlockSpec((BN, H), im_n),
        pl.BlockSpec((BN, 9), im_n),
        pl.BlockSpec((NC, BN, H), im_p),
    ] + [pl.BlockSpec(w.shape, im_w) for w in ws]
    return pl.pallas_call(
        _node_body,
        grid=(N // BN,),
        in_specs=in_specs,
        out_specs=(pl.BlockSpec((BN, H), im_n), pl.BlockSpec((BN, 9), im_n),
                   pl.BlockSpec((BN, 3), im_n), pl.BlockSpec((BN, 3), im_n)),
        out_shape=(jax.ShapeDtypeStruct((N, H), f32),
                   jax.ShapeDtypeStruct((N, 9), f32),
                   jax.ShapeDtypeStruct((N, 3), f32),
                   jax.ShapeDtypeStruct((N, 3), f32)),
        compiler_params=pltpu.CompilerParams(
            dimension_semantics=("arbitrary",)),
    )(nf, nsh, pagg, *ws)


def kernel(node_feat, node_sh, edge_index, diff_pos, diff_vel, radial,
           W_msg1, b_msg1, W_msg2, b_msg2,
           W_pos1, b_pos1, W_pos2, b_pos2,
           W_vel1, b_vel1, W_vel2, b_vel2,
           W_node1, b_node1, W_node2, b_node2,
           W_sh1, b_sh1, W_sh2, b_sh2):
    row = edge_index[0]
    col = edge_index[1]
    shp = jnp.pad(node_sh, ((0, 0), (0, H - 9)))
    dpv = jnp.concatenate(
        [diff_pos, diff_vel, jnp.zeros((E, 2), f32)], axis=1)

    fr, fc, sr, sc_ = _sc_gather(node_feat, shp, row, col)

    w1r = W_msg1[0:H]
    w1c = W_msg1[H:2 * H]
    w1rad = W_msg1[2 * H:2 * H + 16]
    w1ip = jnp.zeros((8, H), f32).at[0:3].set(W_msg1[2 * H + 16:2 * H + 19])
    wh1 = jnp.concatenate([W_pos1, W_vel1, W_sh1], axis=1)
    bh1 = jnp.concatenate([b_pos1, b_vel1, b_sh1])[None, :]
    wh2 = (jnp.zeros((3 * H, 8), f32)
           .at[0:H, 0:2].set(W_pos2)
           .at[H:2 * H, 2:4].set(W_vel2)
           .at[2 * H:3 * H, 4:7].set(W_sh2))
    bh2 = (jnp.zeros((8,), f32)
           .at[0:2].set(b_pos2).at[2:4].set(b_vel2).at[4:7].set(b_sh2))[None, :]

    payload = _tc_edge(fr, fc, sr, sc_, radial, dpv,
                       w1r, w1c, w1rad, w1ip, b_msg1[None, :],
                       W_msg2, b_msg2[None, :], wh1, bh1, wh2, bh2)

    zf = jnp.zeros((NACC, H), f32)
    pagg = _sc_scatter(payload, row, zf)

    wn1a = W_node1[0:H]
    wn1b = W_node1[H:2 * H]
    return _tc_node(node_feat, node_sh, pagg,
                    wn1a, wn1b, b_node1[None, :], W_node2, b_node2[None, :])


# speedup vs baseline: 4.0739x; 1.3583x over previous
import functools

import jax
import jax.numpy as jnp
from jax import lax
from jax.experimental import pallas as pl
from jax.experimental.pallas import tpu as pltpu
from jax.experimental.pallas import tpu_sc as plsc

N = 10000
E = 320000
H = 128

NC = 2
NS = 16
NW = NC * NS
EPW = E // NW
CE = 80
NCHUNK = EPW // CE
NACC = 10240
RPT = NACC // NS
EPT = E // NS
NCHUNK_S = EPT // CE

BE = 2000
BN = 1000

f32 = jnp.float32


def _silu(x):
    return x * (1.0 / (1.0 + jnp.exp(-x)))


def _sc_gather(tpk, row, col):
    mesh = plsc.VectorSubcoreMesh(core_axis_name="c", subcore_axis_name="s")

    @functools.partial(
        pl.kernel,
        out_type=(
            jax.ShapeDtypeStruct((E, H), f32),
            jax.ShapeDtypeStruct((E, H), f32),
        ),
        mesh=mesh,
        scratch_types=(
            pltpu.VMEM((CE,), jnp.int32),
            pltpu.VMEM((CE,), jnp.int32),
            pltpu.VMEM((CE, H), f32),
            pltpu.VMEM((CE, H), f32),
            pltpu.SemaphoreType.DMA,
            pltpu.SemaphoreType.DMA,
        ),
    )
    def gk(tpk_hbm, row_hbm, col_hbm, gr_hbm, gc_hbm,
           idxr_v, idxc_v, rbuf, cbuf, semr, semc):
        cid = lax.axis_index("c")
        sid = lax.axis_index("s")
        wid = sid * NC + cid

        def body(t, carry):
            base = wid * EPW + t * CE
            pltpu.sync_copy(row_hbm.at[pl.ds(base, CE)], idxr_v)
            pltpu.sync_copy(col_hbm.at[pl.ds(base, CE)], idxc_v)
            cr = pltpu.async_copy(tpk_hbm.at[idxr_v], rbuf, semr)
            cc = pltpu.async_copy(tpk_hbm.at[idxc_v], cbuf, semc)
            cr.wait()
            pltpu.sync_copy(rbuf, gr_hbm.at[pl.ds(base, CE)])
            cc.wait()
            pltpu.sync_copy(cbuf, gc_hbm.at[pl.ds(base, CE)])
            return carry

        lax.fori_loop(0, NCHUNK, body, 0)

    return gk(tpk, row, col)


def _sc_scatter(payload, row, zf):
    mesh = plsc.VectorSubcoreMesh(core_axis_name="c", subcore_axis_name="s")

    @functools.partial(
        pl.kernel,
        out_type=jax.ShapeDtypeStruct((NC, NACC, H), f32),
        mesh=mesh,
        scratch_types=(
            pltpu.VMEM((CE,), jnp.int32),
            pltpu.VMEM((CE, H), f32),
            pltpu.VMEM_SHARED((NACC, H), f32),
            pltpu.SemaphoreType.DMA,
        ),
    )
    def sk(p_hbm, row_hbm, zf_hbm, out_hbm, idx_v, pbuf, acc, sem):
        cid = lax.axis_index("c")
        sid = lax.axis_index("s")
        r0 = sid * RPT
        pltpu.sync_copy(zf_hbm.at[pl.ds(r0, RPT)], acc.at[pl.ds(r0, RPT)])
        plsc.subcore_barrier()

        def body(t, carry):
            base = sid * EPT + t * CE
            pltpu.sync_copy(row_hbm.at[pl.ds(base, CE)], idx_v)
            pltpu.sync_copy(p_hbm.at[cid, pl.ds(base, CE)], pbuf)
            pltpu.sync_copy(pbuf, acc.at[idx_v], add=True)
            return carry

        lax.fori_loop(0, NCHUNK_S, body, 0)
        plsc.subcore_barrier()
        pltpu.sync_copy(acc.at[pl.ds(r0, RPT)], out_hbm.at[cid, pl.ds(r0, RPT)])

    return sk(payload, row, zf)


def _unpack(packed):
    u = jax.lax.bitcast_convert_type(packed, jnp.uint32)
    lo = jax.lax.bitcast_convert_type(u << 16, f32)
    hi = jax.lax.bitcast_convert_type(u & jnp.uint32(0xFFFF0000), f32)
    return lo, hi


def _edge_body(gr, gc, rad, dpv,
               wall, w1rad, w_ae, w_ao, b1, w2, b2, wh1, bh1,
               whg1, bg1, whg2, bg2, p1m, p2m, sem, som, ba1,
               p_o):
    bf = jnp.bfloat16

    def dot(a, b):
        return jnp.dot(a, b, preferred_element_type=f32)

    lo_r, hi_r = _unpack(gr[...])
    lo_c, hi_c = _unpack(gc[...])
    plo = (lo_r * lo_c).astype(bf)
    phi = (hi_r * hi_c).astype(bf)
    x = jnp.concatenate([lo_r, hi_r, lo_c, hi_c], axis=1).astype(bf)
    pre = (dot(x, wall[...])
           + dot(rad[...].astype(bf), w1rad[...])
           + dot(plo, w_ae[...]) + dot(phi, w_ao[...])
           + b1[...])
    h = _silu(pre)
    msg = _silu(dot(h.astype(bf), w2[...]) + b2[...])
    gh = _silu(dot(msg.astype(bf), wh1[...]) + bh1[...])
    ghb = gh.astype(bf)
    g1v = dot(ghb, whg1[...]) + bg1[...]
    g2v = dot(ghb, whg2[...]) + bg2[...]
    dlo = (lo_r - lo_c).astype(bf)
    dhi = (hi_r - hi_c).astype(bf)
    dpvb = dpv[...].astype(bf)
    a1v = dot(dpvb, p1m[...]) + dot(dlo, sem[...]) + dot(dhi, som[...]) + ba1[...]
    a2v = dot(dpvb, p2m[...])
    small = g1v * a1v + g2v * a2v
    zpad = jnp.zeros((small.shape[0], H - 16), f32)
    p_o[0] = msg
    p_o[1] = jnp.concatenate([small, zpad], axis=1)


def _tc_edge(gr, gc, rad, dpv, *ws):
    def im_e(i):
        return (i, 0)

    def im_w(i):
        return (0, 0)

    in_specs = [
        pl.BlockSpec((BE, H), im_e),
        pl.BlockSpec((BE, H), im_e),
        pl.BlockSpec((BE, 16), im_e),
        pl.BlockSpec((BE, 8), im_e),
    ] + [pl.BlockSpec(w.shape, im_w) for w in ws]
    return pl.pallas_call(
        _edge_body,
        grid=(E // BE,),
        in_specs=in_specs,
        out_specs=pl.BlockSpec((2, BE, H), lambda i: (0, i, 0)),
        out_shape=jax.ShapeDtypeStruct((2, E, H), f32),
        compiler_params=pltpu.CompilerParams(
            dimension_semantics=("arbitrary",)),
    )(gr, gc, rad, dpv, *ws)


def _node_body(nf, nsh, pr, wn1a, wn1b, bn1, wn2, bn2,
               nf_o, nsh_o, pos_o, vel_o):
    ms = pr[0]
    sm = pr[1]
    inv = 1.0 / jnp.maximum(sm[:, 15:16], 1.0)
    msg_agg = ms * inv
    pos_o[...] = sm[:, 0:3] * inv
    vel_o[...] = sm[:, 3:6] * inv
    nsh_o[...] = nsh[...] + sm[:, 6:15] * inv
    hh = _silu(jnp.dot(nf[...], wn1a[...], preferred_element_type=f32)
               + jnp.dot(msg_agg, wn1b[...], preferred_element_type=f32)
               + bn1[...])
    nf_o[...] = jnp.dot(hh, wn2[...], preferred_element_type=f32) + bn2[...]


def _tc_node(nf, nsh, pagg, *ws):
    def im_n(i):
        return (i, 0)

    def im_p(i):
        return (0, i, 0)

    def im_w(i):
        return (0, 0)

    in_specs = [
        pl.BlockSpec((BN, H), im_n),
        pl.BlockSpec((BN, 9), im_n),
        pl.BlockSpec((NC, BN, H), im_p),
    ] + [pl.BlockSpec(w.shape, im_w) for w in ws]
    return pl.pallas_call(
        _node_body,
        grid=(N // BN,),
        in_specs=in_specs,
        out_specs=(pl.BlockSpec((BN, H), im_n), pl.BlockSpec((BN, 9), im_n),
                   pl.BlockSpec((BN, 3), im_n), pl.BlockSpec((BN, 3), im_n)),
        out_shape=(jax.ShapeDtypeStruct((N, H), f32),
                   jax.ShapeDtypeStruct((N, 9), f32),
                   jax.ShapeDtypeStruct((N, 3), f32),
                   jax.ShapeDtypeStruct((N, 3), f32)),
        compiler_params=pltpu.CompilerParams(
            dimension_semantics=("arbitrary",)),
    )(nf, nsh, pagg, *ws)


def kernel(node_feat, node_sh, edge_index, diff_pos, diff_vel, radial,
           W_msg1, b_msg1, W_msg2, b_msg2,
           W_pos1, b_pos1, W_pos2, b_pos2,
           W_vel1, b_vel1, W_vel2, b_vel2,
           W_node1, b_node1, W_node2, b_node2,
           W_sh1, b_sh1, W_sh2, b_sh2):
    bf = jnp.bfloat16
    row = edge_index[0]
    col = edge_index[1]
    dpv = jnp.concatenate(
        [diff_pos, diff_vel, jnp.zeros((E, 2), f32)], axis=1)

    tb = jnp.concatenate(
        [node_feat, node_sh, jnp.zeros((N, H - 9), f32)], axis=1).astype(bf)
    tpk = jax.lax.bitcast_convert_type(tb.reshape(N, H, 2), f32)

    gr, gc = _sc_gather(tpk, row, col)

    wall = (jnp.zeros((4 * H, H), f32)
            .at[0:64].set(W_msg1[0:H][0::2])
            .at[H:H + 64].set(W_msg1[0:H][1::2])
            .at[2 * H:2 * H + 64].set(W_msg1[H:2 * H][0::2])
            .at[3 * H:3 * H + 64].set(W_msg1[H:2 * H][1::2])).astype(bf)
    w1rad = W_msg1[2 * H:2 * H + 16].astype(bf)
    wip = W_msg1[2 * H + 16:2 * H + 19]
    w_ae = (jnp.zeros((H, H), f32)
            .at[64].set(wip[0]).at[65].set(wip[1])
            .at[66].set(wip[2]).at[67].set(wip[2]).at[68].set(wip[2])
            ).astype(bf)
    w_ao = (jnp.zeros((H, H), f32)
            .at[64].set(wip[1]).at[65].set(wip[1])
            .at[66].set(wip[2]).at[67].set(wip[2])).astype(bf)
    wh1 = jnp.concatenate([W_pos1, W_vel1, W_sh1], axis=1).astype(bf)
    bh1 = jnp.concatenate([b_pos1, b_vel1, b_sh1])[None, :]
    wh2 = (jnp.zeros((3 * H, 8), f32)
           .at[0:H, 0:2].set(W_pos2)
           .at[H:2 * H, 2:4].set(W_vel2)
           .at[2 * H:3 * H, 4:7].set(W_sh2))
    bh2 = (jnp.zeros((8,), f32)
           .at[0:2].set(b_pos2).at[2:4].set(b_vel2).at[4:7].set(b_sh2))[None, :]
    m1 = (jnp.zeros((8, 16), f32)
          .at[0, 0:3].set(1.0).at[2, 3:6].set(1.0).at[4, 6].set(1.0)
          .at[5, 7:10].set(1.0).at[6, 10:15].set(1.0))
    m2 = (jnp.zeros((8, 16), f32)
          .at[1, 0:3].set(1.0).at[3, 3:6].set(1.0))
    whg1 = (wh2 @ m1).astype(bf)
    whg2 = (wh2 @ m2).astype(bf)
    bg1 = (bh2 @ m1).at[0, 15].set(1.0)
    bg2 = bh2 @ m2
    p1m = (jnp.zeros((8, 16), f32)
           .at[0, 0].set(1.0).at[1, 1].set(1.0).at[2, 2].set(1.0)
           .at[3, 3].set(1.0).at[4, 4].set(1.0).at[5, 5].set(1.0)).astype(bf)
    p2m = (jnp.zeros((8, 16), f32)
           .at[3, 0].set(1.0).at[4, 1].set(1.0).at[5, 2].set(1.0)
           .at[0, 3].set(1.0).at[1, 4].set(1.0).at[2, 5].set(1.0)).astype(bf)
    sem = (jnp.zeros((H, 16), f32)
           .at[64, 6].set(1.0).at[65, 8].set(1.0).at[66, 10].set(1.0)
           .at[67, 12].set(1.0).at[68, 14].set(1.0)).astype(bf)
    som = (jnp.zeros((H, 16), f32)
           .at[64, 7].set(1.0).at[65, 9].set(1.0).at[66, 11].set(1.0)
           .at[67, 13].set(1.0)).astype(bf)
    ba1 = jnp.zeros((1, 16), f32).at[0, 15].set(1.0)

    payload = _tc_edge(gr, gc, radial, dpv,
                       wall, w1rad, w_ae, w_ao, b_msg1[None, :],
                       W_msg2.astype(bf), b_msg2[None, :], wh1, bh1,
                       whg1, bg1, whg2, bg2, p1m, p2m, sem, som, ba1)

    zf = jnp.zeros((NACC, H), f32)
    pagg = _sc_scatter(payload, row, zf)

    wn1a = W_node1[0:H]
    wn1b = W_node1[H:2 * H]
    return _tc_node(node_feat, node_sh, pagg,
                    wn1a, wn1b, b_node1[None, :], W_node2, b_node2[None, :])


# speedup vs baseline: 5.0239x; 1.2332x over previous
import functools

import jax
import jax.numpy as jnp
from jax import lax
from jax.experimental import pallas as pl
from jax.experimental.pallas import tpu as pltpu
from jax.experimental.pallas import tpu_sc as plsc

N = 10000
E = 320000
H = 128

NC = 2
NS = 16
NW = NC * NS
EPW = E // NW
CE = 80
NCHUNK = EPW // CE
NACC = 10240
RPT = NACC // NS
EPT = E // NS
NCHUNK_S = EPT // CE

BE = 2000
BN = 1000

f32 = jnp.float32


def _silu(x):
    return x * (1.0 / (1.0 + jnp.exp(-x)))


def _sc_gather(tpk, row, col):
    mesh = plsc.VectorSubcoreMesh(core_axis_name="c", subcore_axis_name="s")

    @functools.partial(
        pl.kernel,
        out_type=(
            jax.ShapeDtypeStruct((E, H), f32),
            jax.ShapeDtypeStruct((E, H), f32),
        ),
        mesh=mesh,
        scratch_types=(
            pltpu.VMEM((2, CE), jnp.int32),
            pltpu.VMEM((2, CE), jnp.int32),
            pltpu.VMEM((2, CE, H), f32),
            pltpu.VMEM((2, CE, H), f32),
            pltpu.SemaphoreType.DMA,
            pltpu.SemaphoreType.DMA,
            pltpu.SemaphoreType.DMA,
            pltpu.SemaphoreType.DMA,
        ),
    )
    def gk(tpk_hbm, row_hbm, col_hbm, gr_hbm, gc_hbm,
           idxr_v, idxc_v, rbuf, cbuf, semr0, semc0, semr1, semc1):
        cid = lax.axis_index("c")
        sid = lax.axis_index("s")
        wid = sid * NC + cid
        sems = ((semr0, semc0), (semr1, semc1))

        def start(b, c):
            semr, semc = sems[b]
            base = wid * EPW + c * CE
            pltpu.sync_copy(row_hbm.at[pl.ds(base, CE)], idxr_v.at[b])
            pltpu.sync_copy(col_hbm.at[pl.ds(base, CE)], idxc_v.at[b])
            pltpu.async_copy(tpk_hbm.at[idxr_v.at[b]], rbuf.at[b], semr)
            pltpu.async_copy(tpk_hbm.at[idxc_v.at[b]], cbuf.at[b], semc)

        def fin(b, c):
            semr, semc = sems[b]
            base = wid * EPW + c * CE
            pltpu.make_async_copy(tpk_hbm.at[idxr_v.at[b]], rbuf.at[b],
                                  semr).wait()
            pltpu.sync_copy(rbuf.at[b], gr_hbm.at[pl.ds(base, CE)])
            pltpu.make_async_copy(tpk_hbm.at[idxc_v.at[b]], cbuf.at[b],
                                  semc).wait()
            pltpu.sync_copy(cbuf.at[b], gc_hbm.at[pl.ds(base, CE)])

        start(0, 0)

        def body(t, carry):
            start(1, 2 * t + 1)
            fin(0, 2 * t)
            start(0, 2 * t + 2)
            fin(1, 2 * t + 1)
            return carry

        lax.fori_loop(0, (NCHUNK - 1) // 2, body, 0)
        fin(0, NCHUNK - 1)

    return gk(tpk, row, col)


def _sc_scatter(payload, row, zf):
    mesh = plsc.VectorSubcoreMesh(core_axis_name="c", subcore_axis_name="s")

    @functools.partial(
        pl.kernel,
        out_type=jax.ShapeDtypeStruct((NC, NACC, H), f32),
        mesh=mesh,
        scratch_types=(
            pltpu.VMEM((2, CE), jnp.int32),
            pltpu.VMEM((2, CE, H), f32),
            pltpu.VMEM_SHARED((NACC, H), f32),
            pltpu.SemaphoreType.DMA,
            pltpu.SemaphoreType.DMA,
            pltpu.SemaphoreType.DMA,
            pltpu.SemaphoreType.DMA,
        ),
    )
    def sk(p_hbm, row_hbm, zf_hbm, out_hbm, idx_v, pbuf,
           acc, semi0, semp0, semi1, semp1):
        cid = lax.axis_index("c")
        sid = lax.axis_index("s")
        r0 = sid * RPT
        pltpu.sync_copy(zf_hbm.at[pl.ds(r0, RPT)], acc.at[pl.ds(r0, RPT)])
        plsc.subcore_barrier()
        sems = ((semi0, semp0), (semi1, semp1))

        def start(b, c):
            semi, semp = sems[b]
            base = sid * EPT + c * CE
            pltpu.async_copy(row_hbm.at[pl.ds(base, CE)], idx_v.at[b], semi)
            pltpu.async_copy(p_hbm.at[cid, pl.ds(base, CE)], pbuf.at[b], semp)

        def fin(b, c):
            semi, semp = sems[b]
            base = sid * EPT + c * CE
            pltpu.make_async_copy(row_hbm.at[pl.ds(base, CE)], idx_v.at[b],
                                  semi).wait()
            pltpu.make_async_copy(p_hbm.at[cid, pl.ds(base, CE)], pbuf.at[b],
                                  semp).wait()
            pltpu.sync_copy(pbuf.at[b], acc.at[idx_v.at[b]], add=True)

        start(0, 0)

        def body(t, carry):
            start(1, 2 * t + 1)
            fin(0, 2 * t)
            start(0, 2 * t + 2)
            fin(1, 2 * t + 1)
            return carry

        lax.fori_loop(0, (NCHUNK_S - 2) // 2, body, 0)
        start(1, NCHUNK_S - 1)
        fin(0, NCHUNK_S - 2)
        fin(1, NCHUNK_S - 1)
        plsc.subcore_barrier()
        pltpu.sync_copy(acc.at[pl.ds(r0, RPT)], out_hbm.at[cid, pl.ds(r0, RPT)])

    return sk(payload, row, zf)


def _unpack(packed):
    u = jax.lax.bitcast_convert_type(packed, jnp.uint32)
    lo = jax.lax.bitcast_convert_type(u << 16, f32)
    hi = jax.lax.bitcast_convert_type(u & jnp.uint32(0xFFFF0000), f32)
    return lo, hi


def _edge_body(gr, gc, rad, dpv,
               wall, w1rad, w_ae, w_ao, b1, w2, b2, wh1, bh1,
               whg1, bg1, whg2, bg2, p1m, p2m, sem, som, ba1,
               p_o):
    bf = jnp.bfloat16

    def dot(a, b):
        return jnp.dot(a, b, preferred_element_type=f32)

    lo_r, hi_r = _unpack(gr[...])
    lo_c, hi_c = _unpack(gc[...])
    plo = (lo_r * lo_c).astype(bf)
    phi = (hi_r * hi_c).astype(bf)
    x = jnp.concatenate([lo_r, hi_r, lo_c, hi_c], axis=1).astype(bf)
    pre = (dot(x, wall[...])
           + dot(rad[...].astype(bf), w1rad[...])
           + dot(plo, w_ae[...]) + dot(phi, w_ao[...])
           + b1[...])
    h = _silu(pre)
    msg = _silu(dot(h.astype(bf), w2[...]) + b2[...])
    gh = _silu(dot(msg.astype(bf), wh1[...]) + bh1[...])
    ghb = gh.astype(bf)
    g1v = dot(ghb, whg1[...]) + bg1[...]
    g2v = dot(ghb, whg2[...]) + bg2[...]
    dlo = (lo_r - lo_c).astype(bf)
    dhi = (hi_r - hi_c).astype(bf)
    dpvb = dpv[...].astype(bf)
    a1v = dot(dpvb, p1m[...]) + dot(dlo, sem[...]) + dot(dhi, som[...]) + ba1[...]
    a2v = dot(dpvb, p2m[...])
    small = g1v * a1v + g2v * a2v
    zpad = jnp.zeros((small.shape[0], H - 16), f32)
    p_o[0] = msg
    p_o[1] = jnp.concatenate([small, zpad], axis=1)


def _tc_edge(gr, gc, rad, dpv, *ws):
    def im_e(i):
        return (i, 0)

    def im_w(i):
        return (0, 0)

    in_specs = [
        pl.BlockSpec((BE, H), im_e),
        pl.BlockSpec((BE, H), im_e),
        pl.BlockSpec((BE, 16), im_e),
        pl.BlockSpec((BE, 8), im_e),
    ] + [pl.BlockSpec(w.shape, im_w) for w in ws]
    return pl.pallas_call(
        _edge_body,
        grid=(E // BE,),
        in_specs=in_specs,
        out_specs=pl.BlockSpec((2, BE, H), lambda i: (0, i, 0)),
        out_shape=jax.ShapeDtypeStruct((2, E, H), f32),
        compiler_params=pltpu.CompilerParams(
            dimension_semantics=("arbitrary",)),
    )(gr, gc, rad, dpv, *ws)


def _node_body(nf, nsh, pr, wn1a, wn1b, bn1, wn2, bn2,
               nf_o, nsh_o, pos_o, vel_o):
    ms = pr[0]
    sm = pr[1]
    inv = 1.0 / jnp.maximum(sm[:, 15:16], 1.0)
    msg_agg = ms * inv
    pos_o[...] = sm[:, 0:3] * inv
    vel_o[...] = sm[:, 3:6] * inv
    nsh_o[...] = nsh[...] + sm[:, 6:15] * inv
    hh = _silu(jnp.dot(nf[...], wn1a[...], preferred_element_type=f32)
               + jnp.dot(msg_agg, wn1b[...], preferred_element_type=f32)
               + bn1[...])
    nf_o[...] = jnp.dot(hh, wn2[...], preferred_element_type=f32) + bn2[...]


def _tc_node(nf, nsh, pagg, *ws):
    def im_n(i):
        return (i, 0)

    def im_p(i):
        return (0, i, 0)

    def im_w(i):
        return (0, 0)

    in_specs = [
        pl.BlockSpec((BN, H), im_n),
        pl.BlockSpec((BN, 9), im_n),
        pl.BlockSpec((NC, BN, H), im_p),
    ] + [pl.BlockSpec(w.shape, im_w) for w in ws]
    return pl.pallas_call(
        _node_body,
        grid=(N // BN,),
        in_specs=in_specs,
        out_specs=(pl.BlockSpec((BN, H), im_n), pl.BlockSpec((BN, 9), im_n),
                   pl.BlockSpec((BN, 3), im_n), pl.BlockSpec((BN, 3), im_n)),
        out_shape=(jax.ShapeDtypeStruct((N, H), f32),
                   jax.ShapeDtypeStruct((N, 9), f32),
                   jax.ShapeDtypeStruct((N, 3), f32),
                   jax.ShapeDtypeStruct((N, 3), f32)),
        compiler_params=pltpu.CompilerParams(
            dimension_semantics=("arbitrary",)),
    )(nf, nsh, pagg, *ws)


def kernel(node_feat, node_sh, edge_index, diff_pos, diff_vel, radial,
           W_msg1, b_msg1, W_msg2, b_msg2,
           W_pos1, b_pos1, W_pos2, b_pos2,
           W_vel1, b_vel1, W_vel2, b_vel2,
           W_node1, b_node1, W_node2, b_node2,
           W_sh1, b_sh1, W_sh2, b_sh2):
    bf = jnp.bfloat16
    row = edge_index[0]
    col = edge_index[1]
    dpv = jnp.concatenate(
        [diff_pos, diff_vel, jnp.zeros((E, 2), f32)], axis=1)

    tb = jnp.concatenate(
        [node_feat, node_sh, jnp.zeros((N, H - 9), f32)], axis=1).astype(bf)
    tpk = jax.lax.bitcast_convert_type(tb.reshape(N, H, 2), f32)

    gr, gc = _sc_gather(tpk, row, col)

    wall = (jnp.zeros((4 * H, H), f32)
            .at[0:64].set(W_msg1[0:H][0::2])
            .at[H:H + 64].set(W_msg1[0:H][1::2])
            .at[2 * H:2 * H + 64].set(W_msg1[H:2 * H][0::2])
            .at[3 * H:3 * H + 64].set(W_msg1[H:2 * H][1::2])).astype(bf)
    w1rad = W_msg1[2 * H:2 * H + 16].astype(bf)
    wip = W_msg1[2 * H + 16:2 * H + 19]
    w_ae = (jnp.zeros((H, H), f32)
            .at[64].set(wip[0]).at[65].set(wip[1])
            .at[66].set(wip[2]).at[67].set(wip[2]).at[68].set(wip[2])
            ).astype(bf)
    w_ao = (jnp.zeros((H, H), f32)
            .at[64].set(wip[1]).at[65].set(wip[1])
            .at[66].set(wip[2]).at[67].set(wip[2])).astype(bf)
    wh1 = jnp.concatenate([W_pos1, W_vel1, W_sh1], axis=1).astype(bf)
    bh1 = jnp.concatenate([b_pos1, b_vel1, b_sh1])[None, :]
    wh2 = (jnp.zeros((3 * H, 8), f32)
           .at[0:H, 0:2].set(W_pos2)
           .at[H:2 * H, 2:4].set(W_vel2)
           .at[2 * H:3 * H, 4:7].set(W_sh2))
    bh2 = (jnp.zeros((8,), f32)
           .at[0:2].set(b_pos2).at[2:4].set(b_vel2).at[4:7].set(b_sh2))[None, :]
    m1 = (jnp.zeros((8, 16), f32)
          .at[0, 0:3].set(1.0).at[2, 3:6].set(1.0).at[4, 6].set(1.0)
          .at[5, 7:10].set(1.0).at[6, 10:15].set(1.0))
    m2 = (jnp.zeros((8, 16), f32)
          .at[1, 0:3].set(1.0).at[3, 3:6].set(1.0))
    whg1 = (wh2 @ m1).astype(bf)
    whg2 = (wh2 @ m2).astype(bf)
    bg1 = (bh2 @ m1).at[0, 15].set(1.0)
    bg2 = bh2 @ m2
    p1m = (jnp.zeros((8, 16), f32)
           .at[0, 0].set(1.0).at[1, 1].set(1.0).at[2, 2].set(1.0)
           .at[3, 3].set(1.0).at[4, 4].set(1.0).at[5, 5].set(1.0)).astype(bf)
    p2m = (jnp.zeros((8, 16), f32)
           .at[3, 0].set(1.0).at[4, 1].set(1.0).at[5, 2].set(1.0)
           .at[0, 3].set(1.0).at[1, 4].set(1.0).at[2, 5].set(1.0)).astype(bf)
    sem = (jnp.zeros((H, 16), f32)
           .at[64, 6].set(1.0).at[65, 8].set(1.0).at[66, 10].set(1.0)
           .at[67, 12].set(1.0).at[68, 14].set(1.0)).astype(bf)
    som = (jnp.zeros((H, 16), f32)
           .at[64, 7].set(1.0).at[65, 9].set(1.0).at[66, 11].set(1.0)
           .at[67, 13].set(1.0)).astype(bf)
    ba1 = jnp.zeros((1, 16), f32).at[0, 15].set(1.0)

    payload = _tc_edge(gr, gc, radial, dpv,
                       wall, w1rad, w_ae, w_ao, b_msg1[None, :],
                       W_msg2.astype(bf), b_msg2[None, :], wh1, bh1,
                       whg1, bg1, whg2, bg2, p1m, p2m, sem, som, ba1)

    zf = jnp.zeros((NACC, H), f32)
    pagg = _sc_scatter(payload, row, zf)

    wn1a = W_node1[0:H]
    wn1b = W_node1[H:2 * H]
    return _tc_node(node_feat, node_sh, pagg,
                    wn1a, wn1b, b_node1[None, :], W_node2, b_node2[None, :])


# speedup vs baseline: 5.4068x; 1.0762x over previous
import functools

import jax
import jax.numpy as jnp
from jax import lax
from jax.experimental import pallas as pl
from jax.experimental.pallas import tpu as pltpu
from jax.experimental.pallas import tpu_sc as plsc

N = 10000
E = 320000
H = 128

NC = 2
NS = 16
NW = NC * NS
EPW = E // NW
CE = 80
NCHUNK = EPW // CE
NACC = 10240
RPT = NACC // NS
EPT = E // NS
NCHUNK_S = EPT // CE

BE = 2000
BN = 1000

f32 = jnp.float32


def _silu(x):
    return x * (1.0 / (1.0 + jnp.exp(-x)))


def _pipe2(start, fin, n):
    start(0, 0)

    def body(t, carry):
        start(1, 2 * t + 1)
        fin(0, 2 * t)
        start(0, 2 * t + 2)
        fin(1, 2 * t + 1)
        return carry

    if n % 2 == 0:
        lax.fori_loop(0, (n - 2) // 2, body, 0)
        start(1, n - 1)
        fin(0, n - 2)
        fin(1, n - 1)
    else:
        lax.fori_loop(0, (n - 1) // 2, body, 0)
        fin(0, n - 1)


def _sc_gather(tpk, row, col, ne, off):
    epw = ne // NW
    nchunk = epw // CE
    mesh = plsc.VectorSubcoreMesh(core_axis_name="c", subcore_axis_name="s")

    @functools.partial(
        pl.kernel,
        out_type=(
            jax.ShapeDtypeStruct((ne, H), f32),
            jax.ShapeDtypeStruct((ne, H), f32),
        ),
        mesh=mesh,
        scratch_types=(
            pltpu.VMEM((2, CE), jnp.int32),
            pltpu.VMEM((2, CE), jnp.int32),
            pltpu.VMEM((2, CE, H), f32),
            pltpu.VMEM((2, CE, H), f32),
            pltpu.SemaphoreType.DMA,
            pltpu.SemaphoreType.DMA,
            pltpu.SemaphoreType.DMA,
            pltpu.SemaphoreType.DMA,
        ),
    )
    def gk(tpk_hbm, row_hbm, col_hbm, gr_hbm, gc_hbm,
           idxr_v, idxc_v, rbuf, cbuf, semr0, semc0, semr1, semc1):
        cid = lax.axis_index("c")
        sid = lax.axis_index("s")
        wid = sid * NC + cid
        sems = ((semr0, semc0), (semr1, semc1))

        def start(b, c):
            semr, semc = sems[b]
            base = wid * epw + c * CE
            pltpu.sync_copy(row_hbm.at[pl.ds(off + base, CE)], idxr_v.at[b])
            pltpu.sync_copy(col_hbm.at[pl.ds(off + base, CE)], idxc_v.at[b])
            pltpu.async_copy(tpk_hbm.at[idxr_v.at[b]], rbuf.at[b], semr)
            pltpu.async_copy(tpk_hbm.at[idxc_v.at[b]], cbuf.at[b], semc)

        def fin(b, c):
            semr, semc = sems[b]
            base = wid * epw + c * CE
            pltpu.make_async_copy(tpk_hbm.at[idxr_v.at[b]], rbuf.at[b],
                                  semr).wait()
            pltpu.sync_copy(rbuf.at[b], gr_hbm.at[pl.ds(base, CE)])
            pltpu.make_async_copy(tpk_hbm.at[idxc_v.at[b]], cbuf.at[b],
                                  semc).wait()
            pltpu.sync_copy(cbuf.at[b], gc_hbm.at[pl.ds(base, CE)])

        _pipe2(start, fin, nchunk)

    return gk(tpk, row, col)


def _sc_scatter(payload, row, zf, ne, off):
    ept = ne // NS
    nchunk_s = ept // CE
    mesh = plsc.VectorSubcoreMesh(core_axis_name="c", subcore_axis_name="s")

    @functools.partial(
        pl.kernel,
        out_type=jax.ShapeDtypeStruct((NC, NACC, H), f32),
        mesh=mesh,
        scratch_types=(
            pltpu.VMEM((2, CE), jnp.int32),
            pltpu.VMEM((2, CE, H), f32),
            pltpu.VMEM_SHARED((NACC, H), f32),
            pltpu.SemaphoreType.DMA,
            pltpu.SemaphoreType.DMA,
            pltpu.SemaphoreType.DMA,
            pltpu.SemaphoreType.DMA,
        ),
    )
    def sk(p_hbm, row_hbm, zf_hbm, out_hbm, idx_v, pbuf,
           acc, semi0, semp0, semi1, semp1):
        cid = lax.axis_index("c")
        sid = lax.axis_index("s")
        r0 = sid * RPT
        pltpu.sync_copy(zf_hbm.at[pl.ds(r0, RPT)], acc.at[pl.ds(r0, RPT)])
        plsc.subcore_barrier()
        sems = ((semi0, semp0), (semi1, semp1))

        def start(b, c):
            semi, semp = sems[b]
            base = sid * ept + c * CE
            pltpu.async_copy(row_hbm.at[pl.ds(off + base, CE)],
                             idx_v.at[b], semi)
            pltpu.async_copy(p_hbm.at[cid, pl.ds(base, CE)], pbuf.at[b], semp)

        def fin(b, c):
            semi, semp = sems[b]
            base = sid * ept + c * CE
            pltpu.make_async_copy(row_hbm.at[pl.ds(off + base, CE)],
                                  idx_v.at[b], semi).wait()
            pltpu.make_async_copy(p_hbm.at[cid, pl.ds(base, CE)], pbuf.at[b],
                                  semp).wait()
            pltpu.sync_copy(pbuf.at[b], acc.at[idx_v.at[b]], add=True)

        _pipe2(start, fin, nchunk_s)
        plsc.subcore_barrier()
        pltpu.sync_copy(acc.at[pl.ds(r0, RPT)], out_hbm.at[cid, pl.ds(r0, RPT)])

    return sk(payload, row, zf)


def _unpack(packed):
    u = jax.lax.bitcast_convert_type(packed, jnp.uint32)
    lo = jax.lax.bitcast_convert_type(u << 16, f32)
    hi = jax.lax.bitcast_convert_type(u & jnp.uint32(0xFFFF0000), f32)
    return lo, hi


def _edge_body(gr, gc, rad, dpv,
               wall, w1rad, w_ae, w_ao, b1, w2, b2, wh1, bh1,
               whg1, bg1, whg2, bg2, p1m, p2m, sem, som, ba1,
               p_o):
    bf = jnp.bfloat16
    n = gr.shape[0]

    def dot(a, b):
        return jnp.dot(a, b, preferred_element_type=f32)

    lo_r, hi_r = _unpack(gr[...])
    lo_c, hi_c = _unpack(gc[...])
    plo = (lo_r * lo_c).astype(bf)
    phi = (hi_r * hi_c).astype(bf)
    x = jnp.concatenate([lo_r, hi_r, lo_c, hi_c], axis=1).astype(bf)
    pre = (dot(x, wall[...])
           + dot(rad[...].astype(bf), w1rad[...])
           + dot(plo, w_ae[...]) + dot(phi, w_ao[...])
           + b1[...])
    h = _silu(pre)
    msg = _silu(dot(h.astype(bf), w2[...]) + b2[...])
    gh = _silu(dot(msg.astype(bf), wh1[...]) + bh1[...])
    ghb = gh.astype(bf)
    g1v = dot(ghb, whg1[...]) + bg1[...]
    g2v = dot(ghb, whg2[...]) + bg2[...]
    dlo = (lo_r - lo_c).astype(bf)
    dhi = (hi_r - hi_c).astype(bf)
    dpvb = dpv[...].astype(bf)
    a1v = (dot(dpvb, p1m[...])
           + dot(dlo, sem[...]) + dot(dhi, som[...]) + ba1[...])
    a2v = dot(dpvb, p2m[...])
    small = g1v * a1v + g2v * a2v
    zpad = jnp.zeros((n, H - 16), f32)
    p_o[0] = msg
    p_o[1] = jnp.concatenate([small, zpad], axis=1)


def _tc_edge(gr, gc, rad, dpv, ne, off, *ws):
    def im_e(i):
        return (i, 0)

    def im_f(i):
        return (off // BE + i, 0)

    def im_w(i):
        return (0, 0)

    in_specs = [
        pl.BlockSpec((BE, H), im_e),
        pl.BlockSpec((BE, H), im_e),
        pl.BlockSpec((BE, 16), im_f),
        pl.BlockSpec((BE, 8), im_f),
    ] + [pl.BlockSpec(w.shape, im_w) for w in ws]
    return pl.pallas_call(
        _edge_body,
        grid=(ne // BE,),
        in_specs=in_specs,
        out_specs=pl.BlockSpec((2, BE, H), lambda i: (0, i, 0)),
        out_shape=jax.ShapeDtypeStruct((2, ne, H), f32),
        compiler_params=pltpu.CompilerParams(
            dimension_semantics=("arbitrary",)),
    )(gr, gc, rad, dpv, *ws)


def _node_body(nf, nsh, pr, qr, wn1a, wn1b, bn1, wn2, bn2,
               nf_o, nsh_o, pos_o, vel_o):
    ms = pr[0] + qr[0]
    sm = pr[1] + qr[1]
    inv = 1.0 / jnp.maximum(sm[:, 15:16], 1.0)
    msg_agg = ms * inv
    pos_o[...] = sm[:, 0:3] * inv
    vel_o[...] = sm[:, 3:6] * inv
    nsh_o[...] = nsh[...] + sm[:, 6:15] * inv
    hh = _silu(jnp.dot(nf[...], wn1a[...], preferred_element_type=f32)
               + jnp.dot(msg_agg, wn1b[...], preferred_element_type=f32)
               + bn1[...])
    nf_o[...] = jnp.dot(hh, wn2[...], preferred_element_type=f32) + bn2[...]


def _tc_node(nf, nsh, pagg, qagg, *ws):
    def im_n(i):
        return (i, 0)

    def im_p(i):
        return (0, i, 0)

    def im_w(i):
        return (0, 0)

    in_specs = [
        pl.BlockSpec((BN, H), im_n),
        pl.BlockSpec((BN, 9), im_n),
        pl.BlockSpec((NC, BN, H), im_p),
        pl.BlockSpec((NC, BN, H), im_p),
    ] + [pl.BlockSpec(w.shape, im_w) for w in ws]
    return pl.pallas_call(
        _node_body,
        grid=(N // BN,),
        in_specs=in_specs,
        out_specs=(pl.BlockSpec((BN, H), im_n), pl.BlockSpec((BN, 9), im_n),
                   pl.BlockSpec((BN, 3), im_n), pl.BlockSpec((BN, 3), im_n)),
        out_shape=(jax.ShapeDtypeStruct((N, H), f32),
                   jax.ShapeDtypeStruct((N, 9), f32),
                   jax.ShapeDtypeStruct((N, 3), f32),
                   jax.ShapeDtypeStruct((N, 3), f32)),
        compiler_params=pltpu.CompilerParams(
            dimension_semantics=("arbitrary",)),
    )(nf, nsh, pagg, qagg, *ws)


def kernel(node_feat, node_sh, edge_index, diff_pos, diff_vel, radial,
           W_msg1, b_msg1, W_msg2, b_msg2,
           W_pos1, b_pos1, W_pos2, b_pos2,
           W_vel1, b_vel1, W_vel2, b_vel2,
           W_node1, b_node1, W_node2, b_node2,
           W_sh1, b_sh1, W_sh2, b_sh2):
    bf = jnp.bfloat16
    row = edge_index[0]
    col = edge_index[1]
    dpv = jnp.concatenate(
        [diff_pos, diff_vel, jnp.zeros((E, 2), f32)], axis=1)

    tb = jnp.concatenate(
        [node_feat, node_sh, jnp.zeros((N, H - 9), f32)], axis=1).astype(bf)
    tpk = jax.lax.bitcast_convert_type(tb.reshape(N, H, 2), f32)

    E1 = 192000
    E2 = E - E1
    gr1, gc1 = _sc_gather(tpk, row, col, E1, 0)
    gr2, gc2 = _sc_gather(tpk, row, col, E2, E1)

    wall = (jnp.zeros((4 * H, H), f32)
            .at[0:64].set(W_msg1[0:H][0::2])
            .at[H:H + 64].set(W_msg1[0:H][1::2])
            .at[2 * H:2 * H + 64].set(W_msg1[H:2 * H][0::2])
            .at[3 * H:3 * H + 64].set(W_msg1[H:2 * H][1::2])).astype(bf)
    w1rad = W_msg1[2 * H:2 * H + 16].astype(bf)
    wip = W_msg1[2 * H + 16:2 * H + 19]
    w_ae = (jnp.zeros((H, H), f32)
            .at[64].set(wip[0]).at[65].set(wip[1])
            .at[66].set(wip[2]).at[67].set(wip[2]).at[68].set(wip[2])
            ).astype(bf)
    w_ao = (jnp.zeros((H, H), f32)
            .at[64].set(wip[1]).at[65].set(wip[1])
            .at[66].set(wip[2]).at[67].set(wip[2])).astype(bf)
    wh1 = jnp.concatenate([W_pos1, W_vel1, W_sh1], axis=1).astype(bf)
    bh1 = jnp.concatenate([b_pos1, b_vel1, b_sh1])[None, :]
    wh2 = (jnp.zeros((3 * H, 8), f32)
           .at[0:H, 0:2].set(W_pos2)
           .at[H:2 * H, 2:4].set(W_vel2)
           .at[2 * H:3 * H, 4:7].set(W_sh2))
    bh2 = (jnp.zeros((8,), f32)
           .at[0:2].set(b_pos2).at[2:4].set(b_vel2).at[4:7].set(b_sh2))[None, :]
    m1 = (jnp.zeros((8, 16), f32)
          .at[0, 0:3].set(1.0).at[2, 3:6].set(1.0).at[4, 6].set(1.0)
          .at[5, 7:10].set(1.0).at[6, 10:15].set(1.0))
    m2 = (jnp.zeros((8, 16), f32)
          .at[1, 0:3].set(1.0).at[3, 3:6].set(1.0))
    whg1 = (wh2 @ m1).astype(bf)
    whg2 = (wh2 @ m2).astype(bf)
    bg1 = (bh2 @ m1).at[0, 15].set(1.0)
    bg2 = bh2 @ m2
    p1m = (jnp.zeros((8, 16), f32)
           .at[0, 0].set(1.0).at[1, 1].set(1.0).at[2, 2].set(1.0)
           .at[3, 3].set(1.0).at[4, 4].set(1.0).at[5, 5].set(1.0)).astype(bf)
    p2m = (jnp.zeros((8, 16), f32)
           .at[3, 0].set(1.0).at[4, 1].set(1.0).at[5, 2].set(1.0)
           .at[0, 3].set(1.0).at[1, 4].set(1.0).at[2, 5].set(1.0)).astype(bf)
    sem = (jnp.zeros((H, 16), f32)
           .at[64, 6].set(1.0).at[65, 8].set(1.0).at[66, 10].set(1.0)
           .at[67, 12].set(1.0).at[68, 14].set(1.0)).astype(bf)
    som = (jnp.zeros((H, 16), f32)
           .at[64, 7].set(1.0).at[65, 9].set(1.0).at[66, 11].set(1.0)
           .at[67, 13].set(1.0)).astype(bf)
    ba1 = jnp.zeros((1, 16), f32).at[0, 15].set(1.0)

    ws_edge = (wall, w1rad, w_ae, w_ao, b_msg1[None, :],
               W_msg2.astype(bf), b_msg2[None, :], wh1, bh1,
               whg1, bg1, whg2, bg2, p1m, p2m, sem, som, ba1)
    payload1 = _tc_edge(gr1, gc1, radial, dpv, E1, 0, *ws_edge)
    payload2 = _tc_edge(gr2, gc2, radial, dpv, E2, E1, *ws_edge)

    zf = jnp.zeros((NACC, H), f32)
    pagg1 = _sc_scatter(payload1, row, zf, E1, 0)
    pagg2 = _sc_scatter(payload2, row, zf, E2, E1)

    wn1a = W_node1[0:H]
    wn1b = W_node1[H:2 * H]
    return _tc_node(node_feat, node_sh, pagg1, pagg2,
                    wn1a, wn1b, b_node1[None, :], W_node2, b_node2[None, :])


# speedup vs baseline: 5.5180x; 1.0206x over previous
import functools

import jax
import jax.numpy as jnp
from jax import lax
from jax.experimental import pallas as pl
from jax.experimental.pallas import tpu as pltpu
from jax.experimental.pallas import tpu_sc as plsc

N = 10000
E = 320000
H = 128

NC = 2
NS = 16
NW = NC * NS
EPW = E // NW
CE = 80
NCHUNK = EPW // CE
NACC = 10240
RPT = NACC // NS
EPT = E // NS
NCHUNK_S = EPT // CE

BE = 2000
BN = 1000

f32 = jnp.float32


def _silu(x):
    return x * (1.0 / (1.0 + jnp.exp(-x)))


def _pipe2(start, fin, n):
    start(0, 0)

    def body(t, carry):
        start(1, 2 * t + 1)
        fin(0, 2 * t)
        start(0, 2 * t + 2)
        fin(1, 2 * t + 1)
        return carry

    if n % 2 == 0:
        lax.fori_loop(0, (n - 2) // 2, body, 0)
        start(1, n - 1)
        fin(0, n - 2)
        fin(1, n - 1)
    else:
        lax.fori_loop(0, (n - 1) // 2, body, 0)
        fin(0, n - 1)


def _sc_gather(tpk, row, col, ne, off):
    epw = ne // NW
    nchunk = epw // CE
    mesh = plsc.VectorSubcoreMesh(core_axis_name="c", subcore_axis_name="s")

    @functools.partial(
        pl.kernel,
        out_type=(
            jax.ShapeDtypeStruct((ne, H), f32),
            jax.ShapeDtypeStruct((ne, H), f32),
        ),
        mesh=mesh,
        scratch_types=(
            pltpu.VMEM((2, CE), jnp.int32),
            pltpu.VMEM((2, CE), jnp.int32),
            pltpu.VMEM((2, CE, H), f32),
            pltpu.VMEM((2, CE, H), f32),
            pltpu.SemaphoreType.DMA,
            pltpu.SemaphoreType.DMA,
            pltpu.SemaphoreType.DMA,
            pltpu.SemaphoreType.DMA,
        ),
    )
    def gk(tpk_hbm, row_hbm, col_hbm, gr_hbm, gc_hbm,
           idxr_v, idxc_v, rbuf, cbuf, semr0, semc0, semr1, semc1):
        cid = lax.axis_index("c")
        sid = lax.axis_index("s")
        wid = sid * NC + cid
        sems = ((semr0, semc0), (semr1, semc1))

        def start(b, c):
            semr, semc = sems[b]
            base = wid * epw + c * CE
            pltpu.sync_copy(row_hbm.at[pl.ds(off + base, CE)], idxr_v.at[b])
            pltpu.sync_copy(col_hbm.at[pl.ds(off + base, CE)], idxc_v.at[b])
            pltpu.async_copy(tpk_hbm.at[idxr_v.at[b]], rbuf.at[b], semr)
            pltpu.async_copy(tpk_hbm.at[idxc_v.at[b]], cbuf.at[b], semc)

        def fin(b, c):
            semr, semc = sems[b]
            base = wid * epw + c * CE
            pltpu.make_async_copy(tpk_hbm.at[idxr_v.at[b]], rbuf.at[b],
                                  semr).wait()
            pltpu.sync_copy(rbuf.at[b], gr_hbm.at[pl.ds(base, CE)])
            pltpu.make_async_copy(tpk_hbm.at[idxc_v.at[b]], cbuf.at[b],
                                  semc).wait()
            pltpu.sync_copy(cbuf.at[b], gc_hbm.at[pl.ds(base, CE)])

        _pipe2(start, fin, nchunk)

    return gk(tpk, row, col)


def _sc_scatter(payload, row, zf, ne, off):
    ept = ne // NS
    nchunk_s = ept // CE
    mesh = plsc.VectorSubcoreMesh(core_axis_name="c", subcore_axis_name="s")

    @functools.partial(
        pl.kernel,
        out_type=jax.ShapeDtypeStruct((NC, NACC, H), f32),
        mesh=mesh,
        scratch_types=(
            pltpu.VMEM((2, CE), jnp.int32),
            pltpu.VMEM((2, CE, H), f32),
            pltpu.VMEM_SHARED((NACC, H), f32),
            pltpu.SemaphoreType.DMA,
            pltpu.SemaphoreType.DMA,
            pltpu.SemaphoreType.DMA,
            pltpu.SemaphoreType.DMA,
        ),
    )
    def sk(p_hbm, row_hbm, zf_hbm, out_hbm, idx_v, pbuf,
           acc, semi0, semp0, semi1, semp1):
        cid = lax.axis_index("c")
        sid = lax.axis_index("s")
        r0 = sid * RPT
        pltpu.sync_copy(zf_hbm.at[pl.ds(r0, RPT)], acc.at[pl.ds(r0, RPT)])
        plsc.subcore_barrier()
        sems = ((semi0, semp0), (semi1, semp1))

        def start(b, c):
            semi, semp = sems[b]
            base = sid * ept + c * CE
            pltpu.async_copy(row_hbm.at[pl.ds(off + base, CE)],
                             idx_v.at[b], semi)
            pltpu.async_copy(p_hbm.at[cid, pl.ds(base, CE)], pbuf.at[b], semp)

        def fin(b, c):
            semi, semp = sems[b]
            base = sid * ept + c * CE
            pltpu.make_async_copy(row_hbm.at[pl.ds(off + base, CE)],
                                  idx_v.at[b], semi).wait()
            pltpu.make_async_copy(p_hbm.at[cid, pl.ds(base, CE)], pbuf.at[b],
                                  semp).wait()
            pltpu.sync_copy(pbuf.at[b], acc.at[idx_v.at[b]], add=True)

        _pipe2(start, fin, nchunk_s)
        plsc.subcore_barrier()
        pltpu.sync_copy(acc.at[pl.ds(r0, RPT)], out_hbm.at[cid, pl.ds(r0, RPT)])

    return sk(payload, row, zf)


def _unpack(packed):
    u = jax.lax.bitcast_convert_type(packed, jnp.uint32)
    lo = jax.lax.bitcast_convert_type(u << 16, f32)
    hi = jax.lax.bitcast_convert_type(u & jnp.uint32(0xFFFF0000), f32)
    return lo, hi


def _edge_body(gr, gc, rad, dpv,
               wall, w1rad, w_ae, w_ao, b1, w2, b2, wh1, bh1,
               whg1, bg1, whg2, bg2, p1m, p2m, sem, som, ba1,
               p_o):
    bf = jnp.bfloat16
    n = gr.shape[0]

    def dot(a, b):
        return jnp.dot(a, b, preferred_element_type=f32)

    lo_r, hi_r = _unpack(gr[...])
    lo_c, hi_c = _unpack(gc[...])
    plo = (lo_r * lo_c).astype(bf)
    phi = (hi_r * hi_c).astype(bf)
    x = jnp.concatenate([lo_r, hi_r, lo_c, hi_c], axis=1).astype(bf)
    pre = (dot(x, wall[...])
           + dot(rad[...].astype(bf), w1rad[...])
           + dot(plo, w_ae[...]) + dot(phi, w_ao[...])
           + b1[...])
    h = _silu(pre)
    msg = _silu(dot(h.astype(bf), w2[...]) + b2[...])
    gh = _silu(dot(msg.astype(bf), wh1[...]) + bh1[...])
    ghb = gh.astype(bf)
    g1v = dot(ghb, whg1[...]) + bg1[...]
    g2v = dot(ghb, whg2[...]) + bg2[...]
    dlo = (lo_r - lo_c).astype(bf)
    dhi = (hi_r - hi_c).astype(bf)
    dpvb = dpv[...].astype(bf)
    a1v = (dot(dpvb, p1m[...])
           + dot(dlo, sem[...]) + dot(dhi, som[...]) + ba1[...])
    a2v = dot(dpvb, p2m[...])
    small = g1v * a1v + g2v * a2v
    zpad = jnp.zeros((n, H - 16), f32)
    p_o[0] = msg
    p_o[1] = jnp.concatenate([small, zpad], axis=1)


def _tc_edge(gr, gc, rad, dpv, ne, off, *ws):
    def im_e(i):
        return (i, 0)

    def im_f(i):
        return (off // BE + i, 0)

    def im_w(i):
        return (0, 0)

    in_specs = [
        pl.BlockSpec((BE, H), im_e),
        pl.BlockSpec((BE, H), im_e),
        pl.BlockSpec((BE, 16), im_f),
        pl.BlockSpec((BE, 8), im_f),
    ] + [pl.BlockSpec(w.shape, im_w) for w in ws]
    return pl.pallas_call(
        _edge_body,
        grid=(ne // BE,),
        in_specs=in_specs,
        out_specs=pl.BlockSpec((2, BE, H), lambda i: (0, i, 0)),
        out_shape=jax.ShapeDtypeStruct((2, ne, H), f32),
        compiler_params=pltpu.CompilerParams(
            dimension_semantics=("arbitrary",)),
    )(gr, gc, rad, dpv, *ws)


def _tc_node(nf, nsh, paggs, *ws):
    k = len(paggs)

    def body(*refs):
        nf_r, nsh_r = refs[0], refs[1]
        prs = refs[2:2 + k]
        wn1a, wn1b, bn1, wn2, bn2 = refs[2 + k:2 + k + 5]
        nf_o, nsh_o, pos_o, vel_o = refs[2 + k + 5:]
        ms = prs[0][0]
        sm = prs[0][1]
        for pr in prs[1:]:
            ms = ms + pr[0]
            sm = sm + pr[1]
        inv = 1.0 / jnp.maximum(sm[:, 15:16], 1.0)
        msg_agg = ms * inv
        pos_o[...] = sm[:, 0:3] * inv
        vel_o[...] = sm[:, 3:6] * inv
        nsh_o[...] = nsh_r[...] + sm[:, 6:15] * inv
        hh = _silu(jnp.dot(nf_r[...], wn1a[...], preferred_element_type=f32)
                   + jnp.dot(msg_agg, wn1b[...], preferred_element_type=f32)
                   + bn1[...])
        nf_o[...] = (jnp.dot(hh, wn2[...], preferred_element_type=f32)
                     + bn2[...])

    def im_n(i):
        return (i, 0)

    def im_p(i):
        return (0, i, 0)

    def im_w(i):
        return (0, 0)

    in_specs = [
        pl.BlockSpec((BN, H), im_n),
        pl.BlockSpec((BN, 9), im_n),
    ] + [pl.BlockSpec((NC, BN, H), im_p)] * k + [
        pl.BlockSpec(w.shape, im_w) for w in ws]
    return pl.pallas_call(
        body,
        grid=(N // BN,),
        in_specs=in_specs,
        out_specs=(pl.BlockSpec((BN, H), im_n), pl.BlockSpec((BN, 9), im_n),
                   pl.BlockSpec((BN, 3), im_n), pl.BlockSpec((BN, 3), im_n)),
        out_shape=(jax.ShapeDtypeStruct((N, H), f32),
                   jax.ShapeDtypeStruct((N, 9), f32),
                   jax.ShapeDtypeStruct((N, 3), f32),
                   jax.ShapeDtypeStruct((N, 3), f32)),
        compiler_params=pltpu.CompilerParams(
            dimension_semantics=("arbitrary",)),
    )(nf, nsh, *paggs, *ws)


def kernel(node_feat, node_sh, edge_index, diff_pos, diff_vel, radial,
           W_msg1, b_msg1, W_msg2, b_msg2,
           W_pos1, b_pos1, W_pos2, b_pos2,
           W_vel1, b_vel1, W_vel2, b_vel2,
           W_node1, b_node1, W_node2, b_node2,
           W_sh1, b_sh1, W_sh2, b_sh2):
    bf = jnp.bfloat16
    row = edge_index[0]
    col = edge_index[1]
    dpv = jnp.concatenate(
        [diff_pos, diff_vel, jnp.zeros((E, 2), f32)], axis=1)

    tb = jnp.concatenate(
        [node_feat, node_sh, jnp.zeros((N, H - 9), f32)], axis=1).astype(bf)
    tpk = jax.lax.bitcast_convert_type(tb.reshape(N, H, 2), f32)

    CHUNKS = ((128000, 0), (128000, 128000), (64000, 256000))
    gs = [_sc_gather(tpk, row, col, ne, off) for ne, off in CHUNKS]

    wall = (jnp.zeros((4 * H, H), f32)
            .at[0:64].set(W_msg1[0:H][0::2])
            .at[H:H + 64].set(W_msg1[0:H][1::2])
            .at[2 * H:2 * H + 64].set(W_msg1[H:2 * H][0::2])
            .at[3 * H:3 * H + 64].set(W_msg1[H:2 * H][1::2])).astype(bf)
    w1rad = W_msg1[2 * H:2 * H + 16].astype(bf)
    wip = W_msg1[2 * H + 16:2 * H + 19]
    w_ae = (jnp.zeros((H, H), f32)
            .at[64].set(wip[0]).at[65].set(wip[1])
            .at[66].set(wip[2]).at[67].set(wip[2]).at[68].set(wip[2])
            ).astype(bf)
    w_ao = (jnp.zeros((H, H), f32)
            .at[64].set(wip[1]).at[65].set(wip[1])
            .at[66].set(wip[2]).at[67].set(wip[2])).astype(bf)
    wh1 = jnp.concatenate([W_pos1, W_vel1, W_sh1], axis=1).astype(bf)
    bh1 = jnp.concatenate([b_pos1, b_vel1, b_sh1])[None, :]
    wh2 = (jnp.zeros((3 * H, 8), f32)
           .at[0:H, 0:2].set(W_pos2)
           .at[H:2 * H, 2:4].set(W_vel2)
           .at[2 * H:3 * H, 4:7].set(W_sh2))
    bh2 = (jnp.zeros((8,), f32)
           .at[0:2].set(b_pos2).at[2:4].set(b_vel2).at[4:7].set(b_sh2))[None, :]
    m1 = (jnp.zeros((8, 16), f32)
          .at[0, 0:3].set(1.0).at[2, 3:6].set(1.0).at[4, 6].set(1.0)
          .at[5, 7:10].set(1.0).at[6, 10:15].set(1.0))
    m2 = (jnp.zeros((8, 16), f32)
          .at[1, 0:3].set(1.0).at[3, 3:6].set(1.0))
    whg1 = (wh2 @ m1).astype(bf)
    whg2 = (wh2 @ m2).astype(bf)
    bg1 = (bh2 @ m1).at[0, 15].set(1.0)
    bg2 = bh2 @ m2
    p1m = (jnp.zeros((8, 16), f32)
           .at[0, 0].set(1.0).at[1, 1].set(1.0).at[2, 2].set(1.0)
           .at[3, 3].set(1.0).at[4, 4].set(1.0).at[5, 5].set(1.0)).astype(bf)
    p2m = (jnp.zeros((8, 16), f32)
           .at[3, 0].set(1.0).at[4, 1].set(1.0).at[5, 2].set(1.0)
           .at[0, 3].set(1.0).at[1, 4].set(1.0).at[2, 5].set(1.0)).astype(bf)
    sem = (jnp.zeros((H, 16), f32)
           .at[64, 6].set(1.0).at[65, 8].set(1.0).at[66, 10].set(1.0)
           .at[67, 12].set(1.0).at[68, 14].set(1.0)).astype(bf)
    som = (jnp.zeros((H, 16), f32)
           .at[64, 7].set(1.0).at[65, 9].set(1.0).at[66, 11].set(1.0)
           .at[67, 13].set(1.0)).astype(bf)
    ba1 = jnp.zeros((1, 16), f32).at[0, 15].set(1.0)

    ws_edge = (wall, w1rad, w_ae, w_ao, b_msg1[None, :],
               W_msg2.astype(bf), b_msg2[None, :], wh1, bh1,
               whg1, bg1, whg2, bg2, p1m, p2m, sem, som, ba1)
    payloads = [
        _tc_edge(g[0], g[1], radial, dpv, ne, off, *ws_edge)
        for g, (ne, off) in zip(gs, CHUNKS)]

    zf = jnp.zeros((NACC, H), f32)
    paggs = [_sc_scatter(p, row, zf, ne, off)
             for p, (ne, off) in zip(payloads, CHUNKS)]

    wn1a = W_node1[0:H]
    wn1b = W_node1[H:2 * H]
    return _tc_node(node_feat, node_sh, paggs,
                    wn1a, wn1b, b_node1[None, :], W_node2, b_node2[None, :])


# speedup vs baseline: 5.5197x; 1.0003x over previous
import functools

import jax
import jax.numpy as jnp
from jax import lax
from jax.experimental import pallas as pl
from jax.experimental.pallas import tpu as pltpu
from jax.experimental.pallas import tpu_sc as plsc

N = 10000
E = 320000
H = 128

NC = 2
NS = 16
NW = NC * NS
CE = 80
NACC = 10240
RPT = NACC // NS

BE = 2000
BN = 1000

f32 = jnp.float32


def _silu(x):
    return x * (1.0 / (1.0 + jnp.exp(-x)))


def _pipe2(start, fin, n):
    start(0, 0)

    def body(t, carry):
        start(1, 2 * t + 1)
        fin(0, 2 * t)
        start(0, 2 * t + 2)
        fin(1, 2 * t + 1)
        return carry

    if n % 2 == 0:
        lax.fori_loop(0, (n - 2) // 2, body, 0)
        start(1, n - 1)
        fin(0, n - 2)
        fin(1, n - 1)
    else:
        lax.fori_loop(0, (n - 1) // 2, body, 0)
        fin(0, n - 1)


def _sc_gather(tpk, row, col, ne, off):
    epw = ne // NW
    nchunk = epw // CE
    mesh = plsc.VectorSubcoreMesh(core_axis_name="c", subcore_axis_name="s")

    @functools.partial(
        pl.kernel,
        out_type=(
            jax.ShapeDtypeStruct((ne, H), f32),
            jax.ShapeDtypeStruct((ne, H), f32),
        ),
        mesh=mesh,
        scratch_types=(
            pltpu.VMEM((2, CE), jnp.int32),
            pltpu.VMEM((2, CE), jnp.int32),
            pltpu.VMEM((2, CE, H), f32),
            pltpu.VMEM((2, CE, H), f32),
            pltpu.SemaphoreType.DMA,
            pltpu.SemaphoreType.DMA,
            pltpu.SemaphoreType.DMA,
            pltpu.SemaphoreType.DMA,
        ),
    )
    def gk(tpk_hbm, row_hbm, col_hbm, gr_hbm, gc_hbm,
           idxr_v, idxc_v, rbuf, cbuf, semr0, semc0, semr1, semc1):
        cid = lax.axis_index("c")
        sid = lax.axis_index("s")
        wid = sid * NC + cid
        sems = ((semr0, semc0), (semr1, semc1))

        def start(b, c):
            semr, semc = sems[b]
            base = wid * epw + c * CE
            pltpu.sync_copy(row_hbm.at[pl.ds(off + base, CE)], idxr_v.at[b])
            pltpu.sync_copy(col_hbm.at[pl.ds(off + base, CE)], idxc_v.at[b])
            pltpu.async_copy(tpk_hbm.at[idxr_v.at[b]], rbuf.at[b], semr)
            pltpu.async_copy(tpk_hbm.at[idxc_v.at[b]], cbuf.at[b], semc)

        def fin(b, c):
            semr, semc = sems[b]
            base = wid * epw + c * CE
            pltpu.make_async_copy(tpk_hbm.at[idxr_v.at[b]], rbuf.at[b],
                                  semr).wait()
            pltpu.sync_copy(rbuf.at[b], gr_hbm.at[pl.ds(base, CE)])
            pltpu.make_async_copy(tpk_hbm.at[idxc_v.at[b]], cbuf.at[b],
                                  semc).wait()
            pltpu.sync_copy(cbuf.at[b], gc_hbm.at[pl.ds(base, CE)])

        _pipe2(start, fin, nchunk)

    return gk(tpk, row, col)


def _sc_scatter(payload, row, zf, ne, off):
    ept = ne // NS
    nchunk_s = ept // CE
    mesh = plsc.VectorSubcoreMesh(core_axis_name="c", subcore_axis_name="s")

    @functools.partial(
        pl.kernel,
        out_type=jax.ShapeDtypeStruct((NC, NACC, H), f32),
        mesh=mesh,
        scratch_types=(
            pltpu.VMEM((2, CE), jnp.int32),
            pltpu.VMEM((2, CE, H), f32),
            pltpu.VMEM_SHARED((NACC, H), f32),
            pltpu.SemaphoreType.DMA,
            pltpu.SemaphoreType.DMA,
            pltpu.SemaphoreType.DMA,
            pltpu.SemaphoreType.DMA,
        ),
    )
    def sk(p_hbm, row_hbm, zf_hbm, out_hbm, idx_v, pbuf,
           acc, semi0, semp0, semi1, semp1):
        cid = lax.axis_index("c")
        sid = lax.axis_index("s")
        r0 = sid * RPT
        pltpu.sync_copy(zf_hbm.at[pl.ds(r0, RPT)], acc.at[pl.ds(r0, RPT)])
        plsc.subcore_barrier()
        sems = ((semi0, semp0), (semi1, semp1))

        def start(b, c):
            semi, semp = sems[b]
            base = sid * ept + c * CE
            pltpu.async_copy(row_hbm.at[pl.ds(off + base, CE)],
                             idx_v.at[b], semi)
            pltpu.async_copy(p_hbm.at[cid, pl.ds(base, CE)], pbuf.at[b], semp)

        def fin(b, c):
            semi, semp = sems[b]
            base = sid * ept + c * CE
            pltpu.make_async_copy(row_hbm.at[pl.ds(off + base, CE)],
                                  idx_v.at[b], semi).wait()
            pltpu.make_async_copy(p_hbm.at[cid, pl.ds(base, CE)], pbuf.at[b],
                                  semp).wait()
            pltpu.sync_copy(pbuf.at[b], acc.at[idx_v.at[b]], add=True)

        _pipe2(start, fin, nchunk_s)
        plsc.subcore_barrier()
        pltpu.sync_copy(acc.at[pl.ds(r0, RPT)], out_hbm.at[cid, pl.ds(r0, RPT)])

    return sk(payload, row, zf)


def _unpack(packed):
    u = jax.lax.bitcast_convert_type(packed, jnp.uint32)
    lo = jax.lax.bitcast_convert_type(u << 16, f32)
    hi = jax.lax.bitcast_convert_type(u & jnp.uint32(0xFFFF0000), f32)
    return lo, hi


def _edge_body(gr, gc, rad, dpv,
               wall, w1rad, w_ae, w_ao, b1, w2, b2, wh1, bh1,
               whg1, bg1, whg2, bg2, p1m, p2m, sem, som, ba1,
               p_o):
    bf = jnp.bfloat16
    n = gr.shape[0]

    def dot(a, b):
        return jnp.dot(a, b, preferred_element_type=f32)

    lo_r, hi_r = _unpack(gr[...])
    lo_c, hi_c = _unpack(gc[...])
    plo = (lo_r * lo_c).astype(bf)
    phi = (hi_r * hi_c).astype(bf)
    x = jnp.concatenate([lo_r, hi_r, lo_c, hi_c], axis=1).astype(bf)
    pre = (dot(x, wall[...])
           + dot(rad[...].astype(bf), w1rad[...])
           + dot(plo, w_ae[...]) + dot(phi, w_ao[...])
           + b1[...])
    h = _silu(pre)
    msg = _silu(dot(h.astype(bf), w2[...]) + b2[...])
    gh = _silu(dot(msg.astype(bf), wh1[...]) + bh1[...])
    ghb = gh.astype(bf)
    g1v = dot(ghb, whg1[...]) + bg1[...]
    g2v = dot(ghb, whg2[...]) + bg2[...]
    dlo = (lo_r - lo_c).astype(bf)
    dhi = (hi_r - hi_c).astype(bf)
    dpvb = dpv[...].astype(bf)
    a1v = (dot(dpvb, p1m[...])
           + dot(dlo, sem[...]) + dot(dhi, som[...]) + ba1[...])
    a2v = dot(dpvb, p2m[...])
    small = g1v * a1v + g2v * a2v
    zpad = jnp.zeros((n, H - 16), f32)
    p_o[0] = msg
    p_o[1] = jnp.concatenate([small, zpad], axis=1)


def _tc_edge(gr, gc, rad, dpv, ne, off, *ws):
    def im_e(i):
        return (i, 0)

    def im_f(i):
        return (off // BE + i, 0)

    def im_w(i):
        return (0, 0)

    in_specs = [
        pl.BlockSpec((BE, H), im_e),
        pl.BlockSpec((BE, H), im_e),
        pl.BlockSpec((BE, 16), im_f),
        pl.BlockSpec((BE, 8), im_f),
    ] + [pl.BlockSpec(w.shape, im_w) for w in ws]
    return pl.pallas_call(
        _edge_body,
        grid=(ne // BE,),
        in_specs=in_specs,
        out_specs=pl.BlockSpec((2, BE, H), lambda i: (0, i, 0)),
        out_shape=jax.ShapeDtypeStruct((2, ne, H), f32),
        compiler_params=pltpu.CompilerParams(
            dimension_semantics=("arbitrary",)),
    )(gr, gc, rad, dpv, *ws)


def _tc_node(nf, nsh, paggs, *ws):
    k = len(paggs)

    def body(*refs):
        nf_r, nsh_r = refs[0], refs[1]
        prs = refs[2:2 + k]
        wn1a, wn1b, bn1, wn2, bn2 = refs[2 + k:2 + k + 5]
        nf_o, nsh_o, pos_o, vel_o = refs[2 + k + 5:]
        ms = prs[0][0]
        sm = prs[0][1]
        for pr in prs[1:]:
            ms = ms + pr[0]
            sm = sm + pr[1]
        inv = 1.0 / jnp.maximum(sm[:, 15:16], 1.0)
        msg_agg = ms * inv
        pos_o[...] = sm[:, 0:3] * inv
        vel_o[...] = sm[:, 3:6] * inv
        nsh_o[...] = nsh_r[...] + sm[:, 6:15] * inv
        hh = _silu(jnp.dot(nf_r[...], wn1a[...], preferred_element_type=f32)
                   + jnp.dot(msg_agg, wn1b[...], preferred_element_type=f32)
                   + bn1[...])
        nf_o[...] = (jnp.dot(hh, wn2[...], preferred_element_type=f32)
                     + bn2[...])

    def im_n(i):
        return (i, 0)

    def im_p(i):
        return (0, i, 0)

    def im_w(i):
        return (0, 0)

    in_specs = [
        pl.BlockSpec((BN, H), im_n),
        pl.BlockSpec((BN, 9), im_n),
    ] + [pl.BlockSpec((NC, BN, H), im_p)] * k + [
        pl.BlockSpec(w.shape, im_w) for w in ws]
    return pl.pallas_call(
        body,
        grid=(N // BN,),
        in_specs=in_specs,
        out_specs=(pl.BlockSpec((BN, H), im_n), pl.BlockSpec((BN, 9), im_n),
                   pl.BlockSpec((BN, 3), im_n), pl.BlockSpec((BN, 3), im_n)),
        out_shape=(jax.ShapeDtypeStruct((N, H), f32),
                   jax.ShapeDtypeStruct((N, 9), f32),
                   jax.ShapeDtypeStruct((N, 3), f32),
                   jax.ShapeDtypeStruct((N, 3), f32)),
        compiler_params=pltpu.CompilerParams(
            dimension_semantics=("arbitrary",)),
    )(nf, nsh, *paggs, *ws)


def kernel(node_feat, node_sh, edge_index, diff_pos, diff_vel, radial,
           W_msg1, b_msg1, W_msg2, b_msg2,
           W_pos1, b_pos1, W_pos2, b_pos2,
           W_vel1, b_vel1, W_vel2, b_vel2,
           W_node1, b_node1, W_node2, b_node2,
           W_sh1, b_sh1, W_sh2, b_sh2):
    bf = jnp.bfloat16
    row = edge_index[0]
    col = edge_index[1]
    dpv = jnp.concatenate(
        [diff_pos, diff_vel, jnp.zeros((E, 2), f32)], axis=1)

    tb = jnp.concatenate(
        [node_feat, node_sh, jnp.zeros((N, H - 9), f32)], axis=1).astype(bf)
    tpk = jax.lax.bitcast_convert_type(tb.reshape(N, H, 2), f32)

    CHUNKS = ((128000, 0), (128000, 128000), (64000, 256000))
    gs = [_sc_gather(tpk, row, col, ne, off) for ne, off in CHUNKS]

    wall = (jnp.zeros((4 * H, H), f32)
            .at[0:64].set(W_msg1[0:H][0::2])
            .at[H:H + 64].set(W_msg1[0:H][1::2])
            .at[2 * H:2 * H + 64].set(W_msg1[H:2 * H][0::2])
            .at[3 * H:3 * H + 64].set(W_msg1[H:2 * H][1::2])).astype(bf)
    w1rad = W_msg1[2 * H:2 * H + 16].astype(bf)
    wip = W_msg1[2 * H + 16:2 * H + 19]
    w_ae = (jnp.zeros((H, H), f32)
            .at[64].set(wip[0]).at[65].set(wip[1])
            .at[66].set(wip[2]).at[67].set(wip[2]).at[68].set(wip[2])
            ).astype(bf)
    w_ao = (jnp.zeros((H, H), f32)
            .at[64].set(wip[1]).at[65].set(wip[1])
            .at[66].set(wip[2]).at[67].set(wip[2])).astype(bf)
    wh1 = jnp.concatenate([W_pos1, W_vel1, W_sh1], axis=1).astype(bf)
    bh1 = jnp.concatenate([b_pos1, b_vel1, b_sh1])[None, :]
    wh2 = (jnp.zeros((3 * H, 8), f32)
           .at[0:H, 0:2].set(W_pos2)
           .at[H:2 * H, 2:4].set(W_vel2)
           .at[2 * H:3 * H, 4:7].set(W_sh2))
    bh2 = (jnp.zeros((8,), f32)
           .at[0:2].set(b_pos2).at[2:4].set(b_vel2).at[4:7].set(b_sh2))[None, :]
    m1 = (jnp.zeros((8, 16), f32)
          .at[0, 0:3].set(1.0).at[2, 3:6].set(1.0).at[4, 6].set(1.0)
          .at[5, 7:10].set(1.0).at[6, 10:15].set(1.0))
    m2 = (jnp.zeros((8, 16), f32)
          .at[1, 0:3].set(1.0).at[3, 3:6].set(1.0))
    whg1 = (wh2 @ m1).astype(bf)
    whg2 = (wh2 @ m2).astype(bf)
    bg1 = (bh2 @ m1).at[0, 15].set(1.0)
    bg2 = bh2 @ m2
    p1m = (jnp.zeros((8, 16), f32)
           .at[0, 0].set(1.0).at[1, 1].set(1.0).at[2, 2].set(1.0)
           .at[3, 3].set(1.0).at[4, 4].set(1.0).at[5, 5].set(1.0)).astype(bf)
    p2m = (jnp.zeros((8, 16), f32)
           .at[3, 0].set(1.0).at[4, 1].set(1.0).at[5, 2].set(1.0)
           .at[0, 3].set(1.0).at[1, 4].set(1.0).at[2, 5].set(1.0)).astype(bf)
    sem = (jnp.zeros((H, 16), f32)
           .at[64, 6].set(1.0).at[65, 8].set(1.0).at[66, 10].set(1.0)
           .at[67, 12].set(1.0).at[68, 14].set(1.0)).astype(bf)
    som = (jnp.zeros((H, 16), f32)
           .at[64, 7].set(1.0).at[65, 9].set(1.0).at[66, 11].set(1.0)
           .at[67, 13].set(1.0)).astype(bf)
    ba1 = jnp.zeros((1, 16), f32).at[0, 15].set(1.0)

    ws_edge = (wall, w1rad, w_ae, w_ao, b_msg1[None, :],
               W_msg2.astype(bf), b_msg2[None, :], wh1, bh1,
               whg1, bg1, whg2, bg2, p1m, p2m, sem, som, ba1)
    payloads = [
        _tc_edge(g[0], g[1], radial, dpv, ne, off, *ws_edge)
        for g, (ne, off) in zip(gs, CHUNKS)]

    zf = jnp.zeros((NACC, H), f32)
    paggs = [_sc_scatter(p, row, zf, ne, off)
             for p, (ne, off) in zip(payloads, CHUNKS)]

    wn1a = W_node1[0:H]
    wn1b = W_node1[H:2 * H]
    return _tc_node(node_feat, node_sh, paggs,
                    wn1a, wn1b, b_node1[None, :], W_node2, b_node2[None, :])
